# baseline (speedup 1.0000x reference)
import jax
import jax.numpy as jnp
from jax import lax
from jax.experimental import pallas as pl
from jax.experimental.pallas import tpu as pltpu
from jax.experimental.pallas import tpu_sc as plsc

B = 4096
L = 50
E = 64
W = 2 * E
NC = 2
NS = 16
NW = NC * NS
BPW = B // NW
CH = 4
NCH = BPW // CH
NG = 4
ROWS = CH * L


def _body(cidx_hbm, coff_hbm, pack_hbm, ctable_hbm, xtable_hbm, out_hbm,
          cidx_v, coff_v, pidx_v, crows_v, buf0, buf1, outbuf, sem0, sem1):
    c = lax.axis_index("c")
    s = lax.axis_index("s")
    wid = s * NC + c
    base = wid * BPW

    pltpu.sync_copy(cidx_hbm.at[pl.ds(base, BPW)], cidx_v)
    pltpu.sync_copy(coff_hbm.at[pl.ds(base, BPW)], coff_v)
    pltpu.sync_copy(pack_hbm.at[pl.ds(base, BPW)], pidx_v)
    pltpu.async_copy(ctable_hbm.at[cidx_v], crows_v, sem0).wait()

    iota = lax.iota(jnp.int32, 16)

    def fire(kchunk, buf, sem):
        for jj in range(CH):
            pltpu.async_copy(
                xtable_hbm.at[pidx_v.at[kchunk * CH + jj, pl.ds(0, L)]],
                buf.at[pl.ds(jj * L, L)], sem)

    def drain(buf, sem):
        for jj in range(CH):
            pltpu.make_async_copy(
                xtable_hbm.at[pidx_v.at[jj, pl.ds(0, L)]],
                buf.at[pl.ds(jj * L, L)], sem).wait()

    def compute(kchunk, buf):
        def jj_body(jj, carry):
            jloc = kchunk * CH + jj
            jvec = jnp.full((16,), jloc, jnp.int32)
            coffb = plsc.load_gather(coff_v, [jvec])
            cv = [plsc.load_gather(crows_v, [jvec, coffb + (i * 16 + iota)])
                  for i in range(E // 16)]
            row0 = jj * L
            outs = [jnp.zeros((16,), jnp.float32) for _ in range(NG)]
            for l in range(L):
                rowv = jnp.full((16,), row0 + l, jnp.int32)
                xoffb = plsc.load_gather(pidx_v, [jvec, iota * 0 + (L + l)])
                p = cv[0] * plsc.load_gather(buf, [rowv, xoffb + iota])
                for i in range(1, E // 16):
                    p = p + cv[i] * plsc.load_gather(
                        buf, [rowv, xoffb + (i * 16 + iota)])
                s_ = jnp.sum(p)
                g, ll = divmod(l, 16)
                outs[g] = jnp.where(iota == ll, s_, outs[g])
            obase = jj * L
            for g in range(NG):
                cols = g * 16 + iota
                plsc.store_scatter(outbuf, [obase + cols], outs[g],
                                   mask=cols < L)
            return carry
        lax.fori_loop(0, CH, jj_body, 0)
        pltpu.sync_copy(outbuf,
                        out_hbm.at[pl.ds((base + kchunk * CH) * L, CH * L)])

    fire(0, buf0, sem0)

    def k2body(k2, carry):
        kc = 2 * k2
        fire(kc + 1, buf1, sem1)
        drain(buf0, sem0)
        compute(kc, buf0)

        @pl.when(kc + 2 < NCH)
        def _():
            fire(kc + 2, buf0, sem0)

        drain(buf1, sem1)
        compute(kc + 1, buf1)
        return carry

    lax.fori_loop(0, NCH // 2, k2body, 0)


_mesh = plsc.VectorSubcoreMesh(core_axis_name="c", subcore_axis_name="s")

_sc_call = pl.kernel(
    _body,
    out_type=jax.ShapeDtypeStruct((B * L,), jnp.float32),
    mesh=_mesh,
    scratch_types=[
        pltpu.VMEM((BPW,), jnp.int32),
        pltpu.VMEM((BPW,), jnp.int32),
        pltpu.VMEM((BPW, W), jnp.int32),
        pltpu.VMEM((BPW, W), jnp.float32),
        pltpu.VMEM((ROWS, W), jnp.float32),
        pltpu.VMEM((ROWS, W), jnp.float32),
        pltpu.VMEM((CH * L,), jnp.float32),
        pltpu.SemaphoreType.DMA,
        pltpu.SemaphoreType.DMA,
    ],
    compiler_params=pltpu.CompilerParams(needs_layout_passes=False, use_tc_tiling_on_sc=True),
)


@jax.jit
def kernel(center, context_negative, center_table, context_table):
    cidx = center.reshape(B)
    chalf = cidx >> 1
    coff = (cidx & 1) * E
    xhalf = context_negative >> 1
    xoff = (context_negative & 1) * E
    pack = jnp.concatenate(
        [xhalf, xoff, jnp.zeros((B, W - 2 * L), jnp.int32)], axis=1)
    t128c = center_table.reshape(100000 // 2, W)
    t128x = context_table.reshape(100000 // 2, W)
    out = _sc_call(chalf, coff, pack, t128c, t128x)
    return out.reshape(B, L)


# speedup vs baseline: 1.3703x; 1.3703x over previous
import jax
import jax.numpy as jnp
from jax import lax
from jax.experimental import pallas as pl
from jax.experimental.pallas import tpu as pltpu
from jax.experimental.pallas import tpu_sc as plsc

B = 4096
L = 50
E = 64
NC = 2
NS = 16
NW = NC * NS
BPW = B // NW
CH = 8
NCH = BPW // CH
NG = 4
PADROWS = CH * L + 16


def _body(cemb_hbm, ctxidx_hbm, xtable_hbm, out_hbm,
          ctxidx_v, crows_v, buf0, buf1, outbuf, sem0, sem1):
    c = lax.axis_index("c")
    s = lax.axis_index("s")
    wid = s * NC + c
    base = wid * BPW

    pltpu.sync_copy(ctxidx_hbm.at[pl.ds(base, BPW)], ctxidx_v)
    pltpu.sync_copy(cemb_hbm.at[pl.ds(base, BPW)], crows_v)

    iota = lax.iota(jnp.int32, 16)

    def fire(kchunk, buf, sem):
        for jj in range(CH):
            pltpu.async_copy(xtable_hbm.at[ctxidx_v.at[kchunk * CH + jj]],
                             buf.at[pl.ds(jj * L, L)], sem)

    def drain(buf, sem):
        for jj in range(CH):
            pltpu.make_async_copy(xtable_hbm.at[ctxidx_v.at[jj]],
                                  buf.at[pl.ds(jj * L, L)], sem).wait()

    def compute(kchunk, buf):
        def jj_body(jj, carry):
            jglob = kchunk * CH + jj
            row0 = jj * L
            cv = [crows_v[jglob, pl.ds(i * 16, 16)] for i in range(E // 16)]
            outs = [jnp.zeros((16,), jnp.float32) for _ in range(NG)]
            for l in range(L):
                row = row0 + l
                p = cv[0] * buf[row, pl.ds(0, 16)]
                for i in range(1, E // 16):
                    p = p + cv[i] * buf[row, pl.ds(i * 16, 16)]
                s = jnp.sum(p)
                g, ll = divmod(l, 16)
                outs[g] = jnp.where(iota == ll, s, outs[g])
            jvec = jnp.full((16,), jj, jnp.int32)
            for g in range(NG):
                cols = g * 16 + iota
                plsc.store_scatter(outbuf, [jvec, cols], outs[g],
                                   mask=cols < L)
            return carry
        lax.fori_loop(0, CH, jj_body, 0)
        pltpu.sync_copy(outbuf, out_hbm.at[pl.ds(base + kchunk * CH, CH)])

    fire(0, buf0, sem0)

    def k2body(k2, carry):
        kc = 2 * k2
        fire(kc + 1, buf1, sem1)
        drain(buf0, sem0)
        compute(kc, buf0)

        @pl.when(kc + 2 < NCH)
        def _():
            fire(kc + 2, buf0, sem0)

        drain(buf1, sem1)
        compute(kc + 1, buf1)
        return carry

    lax.fori_loop(0, NCH // 2, k2body, 0)


_mesh = plsc.VectorSubcoreMesh(core_axis_name="c", subcore_axis_name="s")

_sc_call = pl.kernel(
    _body,
    out_type=jax.ShapeDtypeStruct((B, L), jnp.float32),
    mesh=_mesh,
    scratch_types=[
        pltpu.VMEM((BPW, L), jnp.int32),
        pltpu.VMEM((BPW, E), jnp.float32),
        pltpu.VMEM((PADROWS, E), jnp.float32),
        pltpu.VMEM((PADROWS, E), jnp.float32),
        pltpu.VMEM((CH, L), jnp.float32),
        pltpu.SemaphoreType.DMA,
        pltpu.SemaphoreType.DMA,
    ],
    compiler_params=pltpu.CompilerParams(needs_layout_passes=False, use_tc_tiling_on_sc=False),
)


@jax.jit
def kernel(center, context_negative, center_table, context_table):
    center_embed = jnp.take(center_table, center.reshape(B), axis=0)
    return _sc_call(center_embed, context_negative, context_table)


# speedup vs baseline: 1.3920x; 1.0158x over previous
import jax
import jax.numpy as jnp
from jax import lax
from jax.experimental import pallas as pl
from jax.experimental.pallas import tpu as pltpu
from jax.experimental.pallas import tpu_sc as plsc

B = 4096
L = 50
E = 64
NC = 2
NS = 16
NW = NC * NS
BPW = B // NW
CH = 8
NCH = BPW // CH
NG = 4
PADROWS = CH * L + 16


def _body(cemb_hbm, ctxidx_hbm, xtable_hbm, out_hbm,
          ctxidx_v, crows_v, buf0, buf1, outbuf, sem0, sem1):
    c = lax.axis_index("c")
    s = lax.axis_index("s")
    wid = s * NC + c
    base = wid * BPW

    pltpu.sync_copy(ctxidx_hbm.at[pl.ds(base, BPW)], ctxidx_v)
    pltpu.sync_copy(cemb_hbm.at[pl.ds(base, BPW)], crows_v)

    iota = lax.iota(jnp.int32, 16)

    def fire(kchunk, buf, sem):
        for jj in range(CH):
            pltpu.async_copy(xtable_hbm.at[ctxidx_v.at[kchunk * CH + jj]],
                             buf.at[pl.ds(jj * L, L)], sem)

    def drain(buf, sem):
        for jj in range(CH):
            pltpu.make_async_copy(xtable_hbm.at[ctxidx_v.at[jj]],
                                  buf.at[pl.ds(jj * L, L)], sem).wait()

    def compute(kchunk, buf):
        def jj_body(jj, carry):
            jglob = kchunk * CH + jj
            row0 = jj * L
            cv = [crows_v[jglob, pl.ds(i * 16, 16)] for i in range(E // 16)]
            outs = [jnp.zeros((16,), jnp.float32) for _ in range(NG)]
            for l in range(L):
                row = row0 + l
                p = cv[0] * buf[row, pl.ds(0, 16)]
                for i in range(1, E // 16):
                    p = p + cv[i] * buf[row, pl.ds(i * 16, 16)]
                s = jnp.sum(p)
                g, ll = divmod(l, 16)
                outs[g] = jnp.where(iota == ll, s, outs[g])
            jvec = jnp.full((16,), jj, jnp.int32)
            for g in range(NG):
                cols = g * 16 + iota
                plsc.store_scatter(outbuf, [jvec, cols], outs[g],
                                   mask=cols < L)
            return carry
        lax.fori_loop(0, CH, jj_body, 0)
        pltpu.sync_copy(outbuf, out_hbm.at[pl.ds(base + kchunk * CH, CH)])

    fire(0, buf0, sem0)

    def k2body(k2, carry):
        kc = 2 * k2
        fire(kc + 1, buf1, sem1)
        drain(buf0, sem0)
        compute(kc, buf0)

        @pl.when(kc + 2 < NCH)
        def _():
            fire(kc + 2, buf0, sem0)

        drain(buf1, sem1)
        compute(kc + 1, buf1)
        return carry

    lax.fori_loop(0, NCH // 2, k2body, 0)


_mesh = plsc.VectorSubcoreMesh(core_axis_name="c", subcore_axis_name="s")

_sc_call = pl.kernel(
    _body,
    out_type=jax.ShapeDtypeStruct((B, L), jnp.float32),
    mesh=_mesh,
    scratch_types=[
        pltpu.VMEM((BPW, L), jnp.int32),
        pltpu.VMEM((BPW, E), jnp.float32),
        pltpu.VMEM((PADROWS, E), jnp.float32),
        pltpu.VMEM((PADROWS, E), jnp.float32),
        pltpu.VMEM((CH, L), jnp.float32),
        pltpu.SemaphoreType.DMA,
        pltpu.SemaphoreType.DMA,
    ],
    compiler_params=pltpu.CompilerParams(needs_layout_passes=False, use_tc_tiling_on_sc=False),
)


@jax.jit
def kernel(center, context_negative, center_table, context_table):
    center_embed = jnp.take(center_table, center.reshape(B), axis=0,
                            mode="clip")
    return _sc_call(center_embed, context_negative, context_table)


# speedup vs baseline: 1.3947x; 1.0019x over previous
import jax
import jax.numpy as jnp
from jax import lax
from jax.experimental import pallas as pl
from jax.experimental.pallas import tpu as pltpu
from jax.experimental.pallas import tpu_sc as plsc

B = 4096
L = 50
E = 64
NC = 2
NS = 16
NW = NC * NS
BPW = B // NW
CH = 8
NCH = BPW // CH
NG = 4
PADROWS = CH * L + 16


def _body(ctxidx_hbm, xtable_hbm, cemb_hbm, out_hbm,
          ctxidx_v, crows_v, buf0, buf1, outbuf, sem0, sem1):
    c = lax.axis_index("c")
    s = lax.axis_index("s")
    wid = s * NC + c
    base = wid * BPW

    pltpu.sync_copy(ctxidx_hbm.at[pl.ds(base, BPW)], ctxidx_v)
    pltpu.sync_copy(cemb_hbm.at[pl.ds(base, BPW)], crows_v)

    iota = lax.iota(jnp.int32, 16)

    def fire(kchunk, buf, sem):
        for jj in range(CH):
            pltpu.async_copy(xtable_hbm.at[ctxidx_v.at[kchunk * CH + jj]],
                             buf.at[pl.ds(jj * L, L)], sem)

    def drain(buf, sem):
        for jj in range(CH):
            pltpu.make_async_copy(xtable_hbm.at[ctxidx_v.at[jj]],
                                  buf.at[pl.ds(jj * L, L)], sem).wait()

    def compute(kchunk, buf):
        def jj_body(jj, carry):
            jglob = kchunk * CH + jj
            row0 = jj * L
            cv = [crows_v[jglob, pl.ds(i * 16, 16)] for i in range(E // 16)]
            outs = [jnp.zeros((16,), jnp.float32) for _ in range(NG)]
            for l in range(L):
                row = row0 + l
                p = cv[0] * buf[row, pl.ds(0, 16)]
                for i in range(1, E // 16):
                    p = p + cv[i] * buf[row, pl.ds(i * 16, 16)]
                s = jnp.sum(p)
                g, ll = divmod(l, 16)
                outs[g] = jnp.where(iota == ll, s, outs[g])
            jvec = jnp.full((16,), jj, jnp.int32)
            for g in range(NG):
                cols = g * 16 + iota
                plsc.store_scatter(outbuf, [jvec, cols], outs[g],
                                   mask=cols < L)
            return carry
        lax.fori_loop(0, CH, jj_body, 0)
        pltpu.sync_copy(outbuf, out_hbm.at[pl.ds(base + kchunk * CH, CH)])

    fire(0, buf0, sem0)

    def k2body(k2, carry):
        kc = 2 * k2
        fire(kc + 1, buf1, sem1)
        drain(buf0, sem0)
        compute(kc, buf0)

        @pl.when(kc + 2 < NCH)
        def _():
            fire(kc + 2, buf0, sem0)

        drain(buf1, sem1)
        compute(kc + 1, buf1)
        return carry

    lax.fori_loop(0, NCH // 2, k2body, 0)


_mesh = plsc.VectorSubcoreMesh(core_axis_name="c", subcore_axis_name="s")

_sc_call = pl.kernel(
    _body,
    out_type=jax.ShapeDtypeStruct((B, L), jnp.float32),
    mesh=_mesh,
    scratch_types=[
        pltpu.VMEM((BPW, L), jnp.int32),
        pltpu.VMEM((BPW, E), jnp.float32),
        pltpu.VMEM((PADROWS, E), jnp.float32),
        pltpu.VMEM((PADROWS, E), jnp.float32),
        pltpu.VMEM((CH, L), jnp.float32),
        pltpu.SemaphoreType.DMA,
        pltpu.SemaphoreType.DMA,
    ],
    compiler_params=pltpu.CompilerParams(needs_layout_passes=False, use_tc_tiling_on_sc=False),
)


@jax.jit
def kernel(center, context_negative, center_table, context_table):
    center_embed = jnp.take(center_table, center.reshape(B), axis=0,
                            mode="clip")
    return _sc_call(context_negative, context_table, center_embed)
